# baseline (speedup 1.0000x reference)
import functools

import jax
import jax.numpy as jnp
from jax.experimental import pallas as pl
from jax.experimental.pallas import tpu as pltpu


def _fused_body(rgb_ref, hha_ref, w1_ref, b1_ref, w2d_ref, b2d_ref, out_ref,
                *, inv_hw):
    rgb = rgb_ref[...]
    hha = hha_ref[...]

    sr = jnp.sum(rgb, axis=-1, dtype=jnp.float32)
    sh = jnp.sum(hha, axis=-1, dtype=jnp.float32)
    s = jnp.concatenate([sr, sh], axis=-1)

    h = jnp.dot(s, w1_ref[...], preferred_element_type=jnp.float32)
    h = jnp.maximum(h * inv_hw + b1_ref[...], 0.0)

    d = jnp.dot(h, w2d_ref[...],
                preferred_element_type=jnp.float32) + b2d_ref[...]
    w_rgb = jax.nn.sigmoid(d)

    out_ref[...] = (hha + w_rgb[:, :, None] * (rgb - hha)).astype(out_ref.dtype)


def kernel(rgb, hha, w1, b1, w2, b2):
    assert rgb.shape == hha.shape and rgb.dtype == hha.dtype
    B, C, H, W = rgb.shape
    HW = H * W
    itemsize = jnp.dtype(rgb.dtype).itemsize

    sublanes = 8 * (4 // itemsize)
    S = 1
    if 0 < C < sublanes and sublanes % C == 0 and HW % ((sublanes // C) * 128) == 0:
        S = sublanes // C
    Cf, HWf = C * S, HW // S
    rgbf = rgb.reshape(B, Cf, HWf)
    hhaf = hha.reshape(B, Cf, HWf)

    HID = w1.shape[0]
    w1t = w1.T.astype(jnp.float32)
    w1f = jnp.concatenate([jnp.repeat(w1t[:C], S, axis=0),
                           jnp.repeat(w1t[C:], S, axis=0)], axis=0)
    b1r = b1.astype(jnp.float32).reshape(1, HID)
    w2d = (w2[0] - w2[1]).astype(jnp.float32).reshape(HID, 1)
    b2d = (b2[0] - b2[1]).astype(jnp.float32).reshape(1, 1)

    NB = 1
    grid = (B // NB,)

    def park(shape):
        return pl.BlockSpec(shape, lambda b: (0,) * len(shape))

    body = functools.partial(_fused_body, inv_hw=1.0 / HW)
    outf = pl.pallas_call(
        body,
        out_shape=jax.ShapeDtypeStruct((B, Cf, HWf), rgbf.dtype),
        grid=grid,
        in_specs=[
            pl.BlockSpec((NB, Cf, HWf), lambda b: (b, 0, 0)),
            pl.BlockSpec((NB, Cf, HWf), lambda b: (b, 0, 0)),
            park(w1f.shape), park(b1r.shape), park(w2d.shape), park(b2d.shape),
        ],
        out_specs=pl.BlockSpec((NB, Cf, HWf), lambda b: (b, 0, 0)),
        compiler_params=pltpu.CompilerParams(
            dimension_semantics=("parallel",),
            vmem_limit_bytes=48 * 1024 * 1024),
        cost_estimate=pl.CostEstimate(
            flops=5 * B * Cf * HWf,
            transcendentals=B,
            bytes_accessed=3 * B * Cf * HWf * itemsize),
    )(rgbf, hhaf, w1f, b1r, w2d, b2d)

    return outf.reshape(B, C, H, W)


# speedup vs baseline: 3.1489x; 3.1489x over previous
import functools

import jax
import jax.numpy as jnp
from jax.experimental import pallas as pl
from jax.experimental.pallas import tpu as pltpu


def _fused_body(rgb_ref, hha_ref, w1_ref, b1_ref, w2_ref, b2_ref, out_ref,
                *, inv_hw):
    rgb = rgb_ref[...]
    hha = hha_ref[...]

    sr = jnp.sum(jnp.sum(rgb, axis=-1, dtype=jnp.float32), axis=-1)
    sh = jnp.sum(jnp.sum(hha, axis=-1, dtype=jnp.float32), axis=-1)
    s = jnp.concatenate([sr, sh], axis=-1)

    h = jax.lax.dot_general(s, w1_ref[...], (((1,), (1,)), ((), ())),
                            preferred_element_type=jnp.float32)
    h = jnp.maximum(h * inv_hw + b1_ref[...], 0.0)

    w2d = w2_ref[0:1, :] - w2_ref[1:2, :]
    d = (jnp.sum(h * w2d, axis=-1, keepdims=True)
         + (b2_ref[0:1, 0:1] - b2_ref[0:1, 1:2]))
    w_rgb = jax.nn.sigmoid(d)[:, :, None, None]

    out_ref[...] = (hha + w_rgb * (rgb - hha)).astype(out_ref.dtype)


def kernel(rgb, hha, w1, b1, w2, b2):
    assert rgb.shape == hha.shape and rgb.dtype == hha.dtype
    B, C, H, W = rgb.shape
    HID = w1.shape[0]

    b1r = b1.reshape(1, HID)
    b2r = b2.reshape(1, 2)

    def park(shape):
        return pl.BlockSpec(shape, lambda b: (0,) * len(shape))

    body = functools.partial(_fused_body, inv_hw=1.0 / (H * W))
    return pl.pallas_call(
        body,
        out_shape=jax.ShapeDtypeStruct((B, C, H, W), rgb.dtype),
        grid=(B,),
        in_specs=[
            pl.BlockSpec((1, C, H, W), lambda b: (b, 0, 0, 0)),
            pl.BlockSpec((1, C, H, W), lambda b: (b, 0, 0, 0)),
            park(w1.shape), park(b1r.shape), park(w2.shape), park(b2r.shape),
        ],
        out_specs=pl.BlockSpec((1, C, H, W), lambda b: (b, 0, 0, 0)),
        compiler_params=pltpu.CompilerParams(
            dimension_semantics=("parallel",),
            vmem_limit_bytes=48 * 1024 * 1024),
        cost_estimate=pl.CostEstimate(
            flops=5 * B * C * H * W,
            transcendentals=B,
            bytes_accessed=3 * B * C * H * W * jnp.dtype(rgb.dtype).itemsize),
    )(rgb, hha, w1, b1r, w2, b2r)


# speedup vs baseline: 3.8147x; 1.2114x over previous
import functools

import jax
import jax.numpy as jnp
from jax.experimental import pallas as pl
from jax.experimental.pallas import tpu as pltpu


def _fused_body(rgb_ref, hha_ref, w1_ref, b1_ref, w2_ref, b2_ref, out_ref,
                *, inv_hw):
    rgb = rgb_ref[...]
    hha = hha_ref[...]

    sr = jnp.sum(jnp.sum(rgb, axis=-1, dtype=jnp.float32), axis=-1)
    sh = jnp.sum(jnp.sum(hha, axis=-1, dtype=jnp.float32), axis=-1)
    s = jnp.concatenate([sr, sh], axis=-1)

    h = jax.lax.dot_general(s, w1_ref[...], (((1,), (1,)), ((), ())),
                            preferred_element_type=jnp.float32)
    h = jnp.maximum(h * inv_hw + b1_ref[...], 0.0)

    w2d = w2_ref[0:1, :] - w2_ref[1:2, :]
    d = (jnp.sum(h * w2d, axis=-1, keepdims=True)
         + (b2_ref[0:1, 0:1] - b2_ref[0:1, 1:2]))
    w_rgb = jax.nn.sigmoid(d)[:, :, None, None]

    out_ref[...] = (hha + w_rgb * (rgb - hha)).astype(out_ref.dtype)


def kernel(rgb, hha, w1, b1, w2, b2):
    assert rgb.shape == hha.shape and rgb.dtype == hha.dtype
    B, C, H, W = rgb.shape
    HID = w1.shape[0]

    b1r = b1.reshape(1, HID)
    b2r = b2.reshape(1, 2)

    NB = 2 if B % 2 == 0 else 1

    def park(shape):
        return pl.BlockSpec(shape, lambda b: (0,) * len(shape))

    body = functools.partial(_fused_body, inv_hw=1.0 / (H * W))
    return pl.pallas_call(
        body,
        out_shape=jax.ShapeDtypeStruct((B, C, H, W), rgb.dtype),
        grid=(B // NB,),
        in_specs=[
            pl.BlockSpec((NB, C, H, W), lambda b: (b, 0, 0, 0)),
            pl.BlockSpec((NB, C, H, W), lambda b: (b, 0, 0, 0)),
            park(w1.shape), park(b1r.shape), park(w2.shape), park(b2r.shape),
        ],
        out_specs=pl.BlockSpec((NB, C, H, W), lambda b: (b, 0, 0, 0)),
        compiler_params=pltpu.CompilerParams(
            dimension_semantics=("parallel",),
            vmem_limit_bytes=48 * 1024 * 1024),
        cost_estimate=pl.CostEstimate(
            flops=5 * B * C * H * W,
            transcendentals=B,
            bytes_accessed=3 * B * C * H * W * jnp.dtype(rgb.dtype).itemsize),
    )(rgb, hha, w1, b1r, w2, b2r)


# speedup vs baseline: 3.9745x; 1.0419x over previous
import functools

import jax
import jax.numpy as jnp
from jax.experimental import pallas as pl
from jax.experimental.pallas import tpu as pltpu


def _fused_body(rgb_ref, hha_ref, w1_ref, b1_ref, w2_ref, b2_ref, out_ref,
                *, inv_hw):
    rgb = rgb_ref[...]
    hha = hha_ref[...]

    sr = jnp.sum(jnp.sum(rgb, axis=-1, dtype=jnp.float32), axis=-1)
    sh = jnp.sum(jnp.sum(hha, axis=-1, dtype=jnp.float32), axis=-1)
    s = jnp.concatenate([sr, sh], axis=-1)

    h = jax.lax.dot_general(s, w1_ref[...], (((1,), (1,)), ((), ())),
                            preferred_element_type=jnp.float32)
    h = jnp.maximum(h * inv_hw + b1_ref[...], 0.0)

    w2d = w2_ref[0:1, :] - w2_ref[1:2, :]
    d = (jnp.sum(h * w2d, axis=-1, keepdims=True)
         + (b2_ref[0:1, 0:1] - b2_ref[0:1, 1:2]))
    w_rgb = jax.nn.sigmoid(d)[:, :, None, None]

    out_ref[...] = (hha + w_rgb * (rgb - hha)).astype(out_ref.dtype)


def kernel(rgb, hha, w1, b1, w2, b2):
    assert rgb.shape == hha.shape and rgb.dtype == hha.dtype
    B, C, H, W = rgb.shape
    HID = w1.shape[0]

    b1r = b1.reshape(1, HID)
    b2r = b2.reshape(1, 2)

    NB = 4 if B % 4 == 0 else (2 if B % 2 == 0 else 1)

    def park(shape):
        return pl.BlockSpec(shape, lambda b: (0,) * len(shape))

    body = functools.partial(_fused_body, inv_hw=1.0 / (H * W))
    return pl.pallas_call(
        body,
        out_shape=jax.ShapeDtypeStruct((B, C, H, W), rgb.dtype),
        grid=(B // NB,),
        in_specs=[
            pl.BlockSpec((NB, C, H, W), lambda b: (b, 0, 0, 0)),
            pl.BlockSpec((NB, C, H, W), lambda b: (b, 0, 0, 0)),
            park(w1.shape), park(b1r.shape), park(w2.shape), park(b2r.shape),
        ],
        out_specs=pl.BlockSpec((NB, C, H, W), lambda b: (b, 0, 0, 0)),
        compiler_params=pltpu.CompilerParams(
            dimension_semantics=("parallel",),
            vmem_limit_bytes=48 * 1024 * 1024),
        cost_estimate=pl.CostEstimate(
            flops=5 * B * C * H * W,
            transcendentals=B,
            bytes_accessed=3 * B * C * H * W * jnp.dtype(rgb.dtype).itemsize),
    )(rgb, hha, w1, b1r, w2, b2r)
